# baseline (speedup 1.0000x reference)
import functools

import jax
import jax.numpy as jnp
import numpy as np
from jax import lax
from jax.experimental import pallas as pl
from jax.experimental.pallas import tpu as pltpu
from jax.experimental.pallas import tpu_sc as plsc

SCORE_THR = 0.05
MAX_NUM = 100
N = 5000
C = 20
L = 16
NP = 5008
NV = NP // L
KCAP = 128
OPAD = 128
NCORES = 2
NSUB = 16
BIG = np.int32(1 << 30)

_mesh = plsc.VectorSubcoreMesh(
    core_axis_name="c", subcore_axis_name="s",
    num_cores=NCORES, num_subcores=NSUB)

_f32 = np.float32
_i32 = np.int32


def _wid():
    return lax.axis_index("s") * NCORES + lax.axis_index("c")


def _sload(ref, idx):
    return ref[pl.ds(idx, L)][0]


def _sstore(ref, idx, val, lane0):
    plsc.store_scatter(
        ref, [jnp.full((L,), idx, _i32)], jnp.full((L,), val), mask=lane0)


NB = 256
CH = 96
NTASK = 2 * C


@functools.partial(
    pl.kernel,
    out_type=[jax.ShapeDtypeStruct((C, KCAP), _f32)] * 5
    + [jax.ShapeDtypeStruct((C, KCAP), _i32)],
    mesh=_mesh,
    compiler_params=pltpu.CompilerParams(needs_layout_passes=False, use_tc_tiling_on_sc=False),
    scratch_types=[
        pltpu.VMEM((NP,), _f32),
        pltpu.VMEM((NP + L,), _f32),
        pltpu.VMEM((NP + L,), _f32),
        pltpu.VMEM((NP + L,), _f32),
        pltpu.VMEM((NP + L,), _f32),
        pltpu.VMEM((NP,), _i32),
        pltpu.VMEM((NB * L,), _i32),
        pltpu.VMEM((NSUB * L,), _f32),
        pltpu.VMEM((NP,), _f32),
        pltpu.VMEM((L,), _f32),
        pltpu.VMEM_SHARED((NSUB * L,), _f32),
        pltpu.VMEM((NP + L,), _f32),
        pltpu.VMEM((NP + L,), _i32),
        pltpu.VMEM((NP + L,), _f32),
        pltpu.VMEM((NP + L,), _f32),
        pltpu.VMEM((NP + L,), _f32),
        pltpu.VMEM((NP + L,), _f32),
        pltpu.VMEM((NP + L,), _f32),
        pltpu.VMEM((KCAP,), _f32),
        pltpu.VMEM((KCAP,), _f32),
        pltpu.VMEM((KCAP,), _f32),
        pltpu.VMEM((KCAP,), _f32),
        pltpu.VMEM((KCAP,), _f32),
        pltpu.VMEM((KCAP,), _i32),
        pltpu.VMEM((KCAP,), _f32),
        pltpu.VMEM((KCAP,), _f32),
        pltpu.VMEM((KCAP,), _f32),
        pltpu.VMEM((KCAP,), _f32),
        pltpu.VMEM((KCAP,), _f32),
        pltpu.VMEM((KCAP,), _i32),
    ],
)
def _nms_kernel(s_hbm, bb_hbm,
                osc_hbm, ox1_hbm, oy1_hbm, ox2_hbm, oy2_hbm, on_hbm,
                sv, gx1, gy1, gx2, gy2,
                bkt, hist, mbuf, slab, accb, shm, cs, cidx,
                cx1, cy1, cx2, cy2, car,
                ks, kx1, ky1, kx2, ky2, kn, kbx1, kby1, kbx2, kby2, kbar,
                sel):
    wid = _wid()
    iota = lax.iota(_i32, L)
    lane0 = iota == 0
    ones = jnp.ones((L,), _i32)

    sid = lax.axis_index("s")

    def mslab(j, acc):
        return jnp.maximum(acc, slab[pl.ds(j * L, L)])

    def mtask(t, acc):
        tt = jnp.minimum(sid + t * NSUB, NTASK - 1)
        r = 4 * lax.div(tt, 2) + 2 + lax.rem(tt, 2)
        pltpu.sync_copy(bb_hbm.at[pl.ds(r * NP, NP)], slab)
        return lax.fori_loop(0, NV, mslab, acc)

    acc = lax.fori_loop(0, (NTASK + NSUB - 1) // NSUB, mtask,
                        jnp.full((L,), -1e30, _f32))
    accb[...] = acc
    pltpu.sync_copy(accb, shm.at[pl.ds(sid * L, L)])
    plsc.subcore_barrier()
    pltpu.sync_copy(shm, mbuf)

    def mbody(j, acc):
        return jnp.maximum(acc, mbuf[pl.ds(j * L, L)])

    mxv = lax.fori_loop(0, NSUB, mbody, jnp.full((L,), -1e30, _f32))
    gmax = jnp.max(mxv)

    @pl.when(wid < C)
    def _():
        pltpu.sync_copy(s_hbm.at[pl.ds(wid * NP, NP)], sv)
        base = 4 * wid * NP
        pltpu.sync_copy(bb_hbm.at[pl.ds(base, NP)], gx1.at[pl.ds(0, NP)])
        pltpu.sync_copy(bb_hbm.at[pl.ds(base + NP, NP)],
                        gy1.at[pl.ds(0, NP)])
        pltpu.sync_copy(bb_hbm.at[pl.ds(base + 2 * NP, NP)],
                        gx2.at[pl.ds(0, NP)])
        pltpu.sync_copy(bb_hbm.at[pl.ds(base + 3 * NP, NP)],
                        gy2.at[pl.ds(0, NP)])
        off = wid.astype(_f32) * (gmax + _f32(1.0))

        def hz(j, _):
            hist[pl.ds(j * L, L)] = jnp.zeros((L,), _i32)
            return 0

        lax.fori_loop(0, NB, hz, 0)

        def p1(j, vcnt):
            sl = pl.ds(j * L, L)
            s = sv[sl]
            s = jnp.where(s > _f32(SCORE_THR), s, _f32(-1.0))
            sv[sl] = s
            valid = s > _f32(0.0)
            b = jnp.clip((s * _f32(NB)).astype(_i32), 0, NB - 1)
            bkt[sl] = jnp.where(valid, b, -1)
            plsc.addupdate_scatter(hist, [b * L + iota], ones, mask=valid)
            return vcnt + plsc.all_reduce_population_count(valid)[0]

        vcnt = lax.fori_loop(0, NV, p1, _i32(0))

        def ki(j, _):
            sl = pl.ds(j * L, L)
            ks[sl] = jnp.full((L,), -1.0, _f32)
            kx1[sl] = jnp.zeros((L,), _f32)
            ky1[sl] = jnp.zeros((L,), _f32)
            kx2[sl] = jnp.zeros((L,), _f32)
            ky2[sl] = jnp.zeros((L,), _f32)
            kn[sl] = jnp.zeros((L,), _i32)
            return 0

        lax.fori_loop(0, KCAP // L, ki, 0)

        def outer_cond(st):
            cnt, bp, rem = st
            return (cnt < MAX_NUM) & (rem > 0) & (bp >= 0)

        def outer_body(st):
            cnt, bp, rem = st

            def wcond(ws):
                acc, bptr = ws
                return (acc < CH) & (bptr >= 0)

            def wbody(ws):
                acc, bptr = ws
                cb = jnp.sum(hist[pl.ds(bptr * L, L)])
                return acc + cb, bptr - 1

            acc, bptr = lax.while_loop(wcond, wbody, (_i32(0), bp))
            b_lo = bptr + 1
            rem = rem - acc

            def coll(j, wp):
                sl = pl.ds(j * L, L)
                b = bkt[sl]
                msk = (b >= b_lo) & (b <= bp)
                plsc.store_compressed(cs.at[pl.ds(wp, L)], sv[sl], mask=msk)
                plsc.store_compressed(cidx.at[pl.ds(wp, L)],
                                      j * L + iota, mask=msk)
                return wp + plsc.all_reduce_population_count(msk)[0]

            m_sz = lax.fori_loop(0, NV, coll, _i32(0))
            cs[pl.ds(m_sz, L)] = jnp.full((L,), -1.0, _f32)
            cidx[pl.ds(m_sz, L)] = jnp.zeros((L,), _i32)
            mv = lax.div(m_sz + (L - 1), _i32(L))

            def cg(j, _):
                sl = pl.ds(j * L, L)
                iv = cidx[sl]
                x1 = plsc.load_gather(gx1, [iv]) + off
                y1 = plsc.load_gather(gy1, [iv]) + off
                x2 = plsc.load_gather(gx2, [iv]) + off
                y2 = plsc.load_gather(gy2, [iv]) + off
                cx1[sl] = x1
                cy1[sl] = y1
                cx2[sl] = x2
                cy2[sl] = y2
                car[sl] = (x2 - x1) * (y2 - y1)
                return 0

            lax.fori_loop(0, mv, cg, 0)

            def kchk(k, _):
                bx1 = _sload(kbx1, k)
                by1 = _sload(kby1, k)
                bx2 = _sload(kbx2, k)
                by2 = _sload(kby2, k)
                bar = _sload(kbar, k)

                def kchk_j(j, _2):
                    sl = pl.ds(j * L, L)
                    inter = (jnp.maximum(
                        jnp.minimum(bx2, cx2[sl]) - jnp.maximum(bx1, cx1[sl]),
                        _f32(0.0))
                        * jnp.maximum(
                        jnp.minimum(by2, cy2[sl]) - jnp.maximum(by1, cy1[sl]),
                        _f32(0.0)))
                    den = (bar + car[sl]) - inter + _f32(1e-9)
                    cs[sl] = jnp.where(inter / den > _f32(0.5), _f32(-1.0),
                                       cs[sl])
                    return 0

                lax.fori_loop(0, mv, kchk_j, 0)
                return 0

            lax.fori_loop(0, cnt, kchk, 0)

            def am(j, carry):
                bv, bi = carry
                s = cs[pl.ds(j * L, L)]
                li = j * L + iota
                upd = s > bv
                return jnp.where(upd, s, bv), jnp.where(upd, li, bi)

            bv0, bi0 = lax.fori_loop(
                0, mv, am,
                (jnp.full((L,), -2.0, _f32), jnp.zeros((L,), _i32)))

            def cond(carry):
                cnt2, m, _, _ = carry
                return (cnt2 < MAX_NUM) & (m > _f32(0.0))

            def body(carry):
                cnt2, m, bv, bi = carry
                cand = jnp.where(bv == m, bi, BIG)
                lsel = jnp.full((L,), jnp.min(cand), _i32)
                cntv = jnp.full((L,), cnt2, _i32)
                plsc.store_scatter(sel, [cntv], lsel, mask=lane0)
                bx1 = plsc.load_gather(cx1, [lsel])
                by1 = plsc.load_gather(cy1, [lsel])
                bx2 = plsc.load_gather(cx2, [lsel])
                by2 = plsc.load_gather(cy2, [lsel])
                bar = plsc.load_gather(car, [lsel])

                def sup(j, carry2):
                    bv2, bi2 = carry2
                    sl = pl.ds(j * L, L)
                    s = cs[sl]
                    inter = (jnp.maximum(
                        jnp.minimum(bx2, cx2[sl]) - jnp.maximum(bx1, cx1[sl]),
                        _f32(0.0))
                        * jnp.maximum(
                        jnp.minimum(by2, cy2[sl]) - jnp.maximum(by1, cy1[sl]),
                        _f32(0.0)))
                    den = (bar + car[sl]) - inter + _f32(1e-9)
                    s = jnp.where(inter / den > _f32(0.5), _f32(-1.0), s)
                    cs[sl] = s
                    li = j * L + iota
                    upd = s > bv2
                    return (jnp.where(upd, s, bv2), jnp.where(upd, li, bi2))

                bv2, bi2 = lax.fori_loop(
                    0, mv, sup,
                    (jnp.full((L,), -2.0, _f32), jnp.zeros((L,), _i32)))
                return cnt2 + 1, jnp.max(bv2), bv2, bi2

            cnt1, _, _, _ = lax.while_loop(
                cond, body, (cnt, jnp.max(bv0), bv0, bi0))

            def ep(j, _):
                kv = j * L + iota
                ok = (kv >= cnt) & (kv < cnt1)
                ls = jnp.where(ok, sel[pl.ds(j * L, L)], 0)
                nsl = plsc.load_gather(cidx, [ls])
                plsc.store_scatter(ks, [kv], plsc.load_gather(sv, [nsl]),
                                   mask=ok)
                plsc.store_scatter(kx1, [kv], plsc.load_gather(gx1, [nsl]),
                                   mask=ok)
                plsc.store_scatter(ky1, [kv], plsc.load_gather(gy1, [nsl]),
                                   mask=ok)
                plsc.store_scatter(kx2, [kv], plsc.load_gather(gx2, [nsl]),
                                   mask=ok)
                plsc.store_scatter(ky2, [kv], plsc.load_gather(gy2, [nsl]),
                                   mask=ok)
                plsc.store_scatter(kn, [kv], nsl, mask=ok)
                plsc.store_scatter(kbx1, [kv], plsc.load_gather(cx1, [ls]),
                                   mask=ok)
                plsc.store_scatter(kby1, [kv], plsc.load_gather(cy1, [ls]),
                                   mask=ok)
                plsc.store_scatter(kbx2, [kv], plsc.load_gather(cx2, [ls]),
                                   mask=ok)
                plsc.store_scatter(kby2, [kv], plsc.load_gather(cy2, [ls]),
                                   mask=ok)
                plsc.store_scatter(kbar, [kv], plsc.load_gather(car, [ls]),
                                   mask=ok)
                return 0

            lax.fori_loop(lax.div(cnt, _i32(L)),
                          lax.div(cnt1 + (L - 1), _i32(L)), ep, 0)
            return cnt1, bptr, rem

        lax.while_loop(outer_cond, outer_body, (_i32(0), _i32(NB - 1), vcnt))

        pltpu.sync_copy(ks, osc_hbm.at[wid])
        pltpu.sync_copy(kx1, ox1_hbm.at[wid])
        pltpu.sync_copy(ky1, oy1_hbm.at[wid])
        pltpu.sync_copy(kx2, ox2_hbm.at[wid])
        pltpu.sync_copy(ky2, oy2_hbm.at[wid])
        pltpu.sync_copy(kn, on_hbm.at[wid])


@functools.partial(
    pl.kernel,
    out_type=[jax.ShapeDtypeStruct((OPAD * 5,), _f32),
              jax.ShapeDtypeStruct((OPAD,), _i32)],
    mesh=_mesh,
    compiler_params=pltpu.CompilerParams(needs_layout_passes=False, use_tc_tiling_on_sc=False),
    scratch_types=[
        pltpu.VMEM((C * KCAP + L,), _f32),
        pltpu.VMEM((C * KCAP + L,), _f32),
        pltpu.VMEM((C * KCAP + L,), _f32),
        pltpu.VMEM((C * KCAP + L,), _f32),
        pltpu.VMEM((C * KCAP + L,), _f32),
        pltpu.VMEM((C * KCAP + L,), _i32),
        pltpu.VMEM((2 * L,), _f32),
        pltpu.VMEM((2 * L,), _i32),
        pltpu.SMEM((2 * L,), _i32),
        pltpu.VMEM((OPAD * 5,), _f32),
        pltpu.VMEM((OPAD,), _i32),
        pltpu.VMEM((OPAD,), _i32),
    ],
)
def _merge_kernel(sc_hbm, x1_hbm, y1_hbm, x2_hbm, y2_hbm, n_hbm,
                  dets_hbm, labels_hbm,
                  vsc, vx1, vy1, vx2, vy2, vn, hs, hn, hp, dv, lv, sb):
    wid = _wid()
    iota = lax.iota(_i32, L)
    lane0 = iota == 0

    @pl.when(wid == 0)
    def _():
        pltpu.sync_copy(sc_hbm, vsc.at[pl.ds(0, C * KCAP)])
        pltpu.sync_copy(x1_hbm, vx1.at[pl.ds(0, C * KCAP)])
        pltpu.sync_copy(y1_hbm, vy1.at[pl.ds(0, C * KCAP)])
        pltpu.sync_copy(x2_hbm, vx2.at[pl.ds(0, C * KCAP)])
        pltpu.sync_copy(y2_hbm, vy2.at[pl.ds(0, C * KCAP)])
        pltpu.sync_copy(n_hbm, vn.at[pl.ds(0, C * KCAP)])

        for half in range(2):
            cv = iota + half * L
            cidx = jnp.minimum(cv, C - 1) * KCAP
            h = plsc.load_gather(vsc, [cidx])
            hs[pl.ds(half * L, L)] = jnp.where(cv < C, h, _f32(-1.0))
            nh = plsc.load_gather(vn, [cidx])
            hn[pl.ds(half * L, L)] = jnp.where(cv < C, nh, _i32(0))

        def pinit(c, _):
            hp[c] = _i32(0)
            return 0

        lax.fori_loop(0, 2 * L, pinit, 0)

        def sinit(k, _):
            sb[pl.ds(k * L, L)] = jnp.full((L,), -1, _i32)
            return 0

        lax.fori_loop(0, OPAD // L, sinit, 0)

        def mbody(k, _):
            h1 = hs[pl.ds(0, L)]
            h2 = hs[pl.ds(L, L)]
            m = jnp.maximum(jnp.max(h1), jnp.max(h2))

            @pl.when(m > _f32(0.0))
            def _():
                n1 = hn[pl.ds(0, L)]
                n2 = hn[pl.ds(L, L)]
                fi1 = jnp.where(h1 == m, n1 * C + iota, BIG)
                fi2 = jnp.where(h2 == m, n2 * C + (iota + L), BIG)
                fi = jnp.minimum(jnp.min(fi1), jnp.min(fi2))
                csel = lax.rem(fi, _i32(C))
                p = hp[csel]
                base = csel * KCAP + p
                _sstore(sb, k, base, lane0)
                pn = p + 1
                hp[csel] = pn
                pc = jnp.minimum(pn, KCAP - 1)
                nxt = _sload(vsc, csel * KCAP + pc)
                _sstore(hs, csel, jnp.where(pn > KCAP - 1, _f32(-1.0), nxt),
                        lane0)
                _sstore(hn, csel, _sload(vn, csel * KCAP + pc), lane0)

            return 0

        lax.fori_loop(0, MAX_NUM, mbody, 0)

        allm = iota >= 0

        def epi(j, _):
            kv = j * L + iota
            b = sb[pl.ds(j * L, L)]
            ok = b >= 0
            bc = jnp.where(ok, b, 0)
            zero = jnp.zeros((L,), _f32)
            plsc.store_scatter(
                dv, [kv * 5],
                jnp.where(ok, plsc.load_gather(vx1, [bc]), zero), mask=allm)
            plsc.store_scatter(
                dv, [kv * 5 + 1],
                jnp.where(ok, plsc.load_gather(vy1, [bc]), zero), mask=allm)
            plsc.store_scatter(
                dv, [kv * 5 + 2],
                jnp.where(ok, plsc.load_gather(vx2, [bc]), zero), mask=allm)
            plsc.store_scatter(
                dv, [kv * 5 + 3],
                jnp.where(ok, plsc.load_gather(vy2, [bc]), zero), mask=allm)
            plsc.store_scatter(
                dv, [kv * 5 + 4],
                jnp.where(ok, plsc.load_gather(vsc, [bc]), zero), mask=allm)
            lv[pl.ds(j * L, L)] = jnp.where(
                ok, lax.shift_right_logical(bc, 7), -1)
            return 0

        lax.fori_loop(0, OPAD // L, epi, 0)

        pltpu.sync_copy(dv, dets_hbm)
        pltpu.sync_copy(lv, labels_hbm)


def kernel(multi_bboxes, multi_scores):
    pad = ((0, 0), (0, NP - N))
    bbt = jnp.pad(multi_bboxes.T, pad)
    st = jnp.pad(multi_scores.T[:C], pad)
    k_sc, k_x1, k_y1, k_x2, k_y2, k_n = _nms_kernel(
        st.reshape(-1), bbt.reshape(-1))
    dets_pad, labels_pad = _merge_kernel(
        k_sc.reshape(-1), k_x1.reshape(-1), k_y1.reshape(-1),
        k_x2.reshape(-1), k_y2.reshape(-1), k_n.reshape(-1))
    dets = dets_pad.reshape(OPAD, 5)[:MAX_NUM]
    labels = labels_pad[:MAX_NUM]
    return dets, labels


# speedup vs baseline: 1.1538x; 1.1538x over previous
import functools

import jax
import jax.numpy as jnp
import numpy as np
from jax import lax
from jax.experimental import pallas as pl
from jax.experimental.pallas import tpu as pltpu
from jax.experimental.pallas import tpu_sc as plsc

SCORE_THR = 0.05
MAX_NUM = 100
N = 5000
C = 20
L = 16
NP = 5008
NV = NP // L
KCAP = 128
OPAD = 128
NCORES = 2
NSUB = 16
BIG = np.int32(1 << 30)

_mesh = plsc.VectorSubcoreMesh(
    core_axis_name="c", subcore_axis_name="s",
    num_cores=NCORES, num_subcores=NSUB)

_f32 = np.float32
_i32 = np.int32


def _wid():
    return lax.axis_index("s") * NCORES + lax.axis_index("c")


def _sload(ref, idx):
    return ref[pl.ds(idx, L)][0]


def _sstore(ref, idx, val, lane0):
    plsc.store_scatter(
        ref, [jnp.full((L,), idx, _i32)], jnp.full((L,), val), mask=lane0)


NB = 256
CH = 128
NTASK = 2 * C


@functools.partial(
    pl.kernel,
    out_type=[jax.ShapeDtypeStruct((C, KCAP), _f32)] * 5
    + [jax.ShapeDtypeStruct((C, KCAP), _i32)],
    mesh=_mesh,
    compiler_params=pltpu.CompilerParams(needs_layout_passes=False, use_tc_tiling_on_sc=False),
    scratch_types=[
        pltpu.VMEM((NP,), _f32),
        pltpu.VMEM((NP + L,), _f32),
        pltpu.VMEM((NP + L,), _f32),
        pltpu.VMEM((NP + L,), _f32),
        pltpu.VMEM((NP + L,), _f32),
        pltpu.VMEM((NP,), _i32),
        pltpu.VMEM((NB * L,), _i32),
        pltpu.VMEM((NSUB * L,), _f32),
        pltpu.VMEM((NP,), _f32),
        pltpu.VMEM((L,), _f32),
        pltpu.VMEM_SHARED((NSUB * L,), _f32),
        pltpu.VMEM((NP + L,), _f32),
        pltpu.VMEM((NP + L,), _i32),
        pltpu.VMEM((NP + L,), _f32),
        pltpu.VMEM((NP + L,), _f32),
        pltpu.VMEM((NP + L,), _f32),
        pltpu.VMEM((NP + L,), _f32),
        pltpu.VMEM((NP + L,), _f32),
        pltpu.VMEM((KCAP,), _f32),
        pltpu.VMEM((KCAP,), _f32),
        pltpu.VMEM((KCAP,), _f32),
        pltpu.VMEM((KCAP,), _f32),
        pltpu.VMEM((KCAP,), _f32),
        pltpu.VMEM((KCAP,), _i32),
        pltpu.VMEM((KCAP,), _f32),
        pltpu.VMEM((KCAP,), _f32),
        pltpu.VMEM((KCAP,), _f32),
        pltpu.VMEM((KCAP,), _f32),
        pltpu.VMEM((KCAP,), _f32),
        pltpu.VMEM((KCAP,), _i32),
    ],
)
def _nms_kernel(s_hbm, bb_hbm,
                osc_hbm, ox1_hbm, oy1_hbm, ox2_hbm, oy2_hbm, on_hbm,
                sv, gx1, gy1, gx2, gy2,
                bkt, hist, mbuf, slab, accb, shm, cs, cidx,
                cx1, cy1, cx2, cy2, car,
                ks, kx1, ky1, kx2, ky2, kn, kbx1, kby1, kbx2, kby2, kbar,
                sel):
    wid = _wid()
    iota = lax.iota(_i32, L)
    lane0 = iota == 0
    ones = jnp.ones((L,), _i32)

    sid = lax.axis_index("s")

    def mslab(j, acc):
        return jnp.maximum(acc, slab[pl.ds(j * L, L)])

    def mtask(t, acc):
        tt = jnp.minimum(sid + t * NSUB, NTASK - 1)
        r = 4 * lax.div(tt, 2) + 2 + lax.rem(tt, 2)
        pltpu.sync_copy(bb_hbm.at[pl.ds(r * NP, NP)], slab)
        return lax.fori_loop(0, NV, mslab, acc)

    acc = lax.fori_loop(0, (NTASK + NSUB - 1) // NSUB, mtask,
                        jnp.full((L,), -1e30, _f32))
    accb[...] = acc
    pltpu.sync_copy(accb, shm.at[pl.ds(sid * L, L)])
    plsc.subcore_barrier()
    pltpu.sync_copy(shm, mbuf)

    def mbody(j, acc):
        return jnp.maximum(acc, mbuf[pl.ds(j * L, L)])

    mxv = lax.fori_loop(0, NSUB, mbody, jnp.full((L,), -1e30, _f32))
    gmax = jnp.max(mxv)

    @pl.when(wid < C)
    def _():
        pltpu.sync_copy(s_hbm.at[pl.ds(wid * NP, NP)], sv)
        base = 4 * wid * NP
        pltpu.sync_copy(bb_hbm.at[pl.ds(base, NP)], gx1.at[pl.ds(0, NP)])
        pltpu.sync_copy(bb_hbm.at[pl.ds(base + NP, NP)],
                        gy1.at[pl.ds(0, NP)])
        pltpu.sync_copy(bb_hbm.at[pl.ds(base + 2 * NP, NP)],
                        gx2.at[pl.ds(0, NP)])
        pltpu.sync_copy(bb_hbm.at[pl.ds(base + 3 * NP, NP)],
                        gy2.at[pl.ds(0, NP)])
        off = wid.astype(_f32) * (gmax + _f32(1.0))

        def hz(j, _):
            hist[pl.ds(j * L, L)] = jnp.zeros((L,), _i32)
            return 0

        lax.fori_loop(0, NB, hz, 0)

        def p1(j, vcnt):
            sl = pl.ds(j * L, L)
            s = sv[sl]
            s = jnp.where(s > _f32(SCORE_THR), s, _f32(-1.0))
            sv[sl] = s
            valid = s > _f32(0.0)
            b = jnp.clip((s * _f32(NB)).astype(_i32), 0, NB - 1)
            bkt[sl] = jnp.where(valid, b, -1)
            plsc.addupdate_scatter(hist, [b * L + iota], ones, mask=valid)
            return vcnt + plsc.all_reduce_population_count(valid)[0]

        vcnt = lax.fori_loop(0, NV, p1, _i32(0))

        def ki(j, _):
            sl = pl.ds(j * L, L)
            ks[sl] = jnp.full((L,), -1.0, _f32)
            kx1[sl] = jnp.zeros((L,), _f32)
            ky1[sl] = jnp.zeros((L,), _f32)
            kx2[sl] = jnp.zeros((L,), _f32)
            ky2[sl] = jnp.zeros((L,), _f32)
            kn[sl] = jnp.zeros((L,), _i32)
            return 0

        lax.fori_loop(0, KCAP // L, ki, 0)

        def outer_cond(st):
            cnt, bp, rem = st
            return (cnt < MAX_NUM) & (rem > 0) & (bp >= 0)

        def outer_body(st):
            cnt, bp, rem = st

            def wcond(ws):
                acc, bptr = ws
                return (acc < CH) & (bptr >= 0)

            def wbody(ws):
                acc, bptr = ws
                cb = jnp.sum(hist[pl.ds(bptr * L, L)])
                return acc + cb, bptr - 1

            acc, bptr = lax.while_loop(wcond, wbody, (_i32(0), bp))
            b_lo = bptr + 1
            rem = rem - acc

            def coll(j, wp):
                sl = pl.ds(j * L, L)
                b = bkt[sl]
                msk = (b >= b_lo) & (b <= bp)
                plsc.store_compressed(cs.at[pl.ds(wp, L)], sv[sl], mask=msk)
                plsc.store_compressed(cidx.at[pl.ds(wp, L)],
                                      j * L + iota, mask=msk)
                return wp + plsc.all_reduce_population_count(msk)[0]

            m_sz = lax.fori_loop(0, NV, coll, _i32(0))
            cs[pl.ds(m_sz, L)] = jnp.full((L,), -1.0, _f32)
            cidx[pl.ds(m_sz, L)] = jnp.zeros((L,), _i32)
            mv = lax.div(m_sz + (L - 1), _i32(L))

            def cg(j, _):
                sl = pl.ds(j * L, L)
                iv = cidx[sl]
                x1 = plsc.load_gather(gx1, [iv]) + off
                y1 = plsc.load_gather(gy1, [iv]) + off
                x2 = plsc.load_gather(gx2, [iv]) + off
                y2 = plsc.load_gather(gy2, [iv]) + off
                cx1[sl] = x1
                cy1[sl] = y1
                cx2[sl] = x2
                cy2[sl] = y2
                car[sl] = (x2 - x1) * (y2 - y1)
                return 0

            lax.fori_loop(0, mv, cg, 0)

            def kchk(k, _):
                bx1 = _sload(kbx1, k)
                by1 = _sload(kby1, k)
                bx2 = _sload(kbx2, k)
                by2 = _sload(kby2, k)
                bar = _sload(kbar, k)

                def kchk_j(j, _2):
                    sl = pl.ds(j * L, L)
                    inter = (jnp.maximum(
                        jnp.minimum(bx2, cx2[sl]) - jnp.maximum(bx1, cx1[sl]),
                        _f32(0.0))
                        * jnp.maximum(
                        jnp.minimum(by2, cy2[sl]) - jnp.maximum(by1, cy1[sl]),
                        _f32(0.0)))
                    den = (bar + car[sl]) - inter + _f32(1e-9)
                    cs[sl] = jnp.where(inter / den > _f32(0.5), _f32(-1.0),
                                       cs[sl])
                    return 0

                lax.fori_loop(0, mv, kchk_j, 0)
                return 0

            lax.fori_loop(0, cnt, kchk, 0)

            def am(j, carry):
                bv, bi = carry
                s = cs[pl.ds(j * L, L)]
                li = j * L + iota
                upd = s > bv
                return jnp.where(upd, s, bv), jnp.where(upd, li, bi)

            bv0, bi0 = lax.fori_loop(
                0, mv, am,
                (jnp.full((L,), -2.0, _f32), jnp.zeros((L,), _i32)))

            def cond(carry):
                cnt2, m, _, _ = carry
                return (cnt2 < MAX_NUM) & (m > _f32(0.0))

            def body(carry):
                cnt2, m, bv, bi = carry
                cand = jnp.where(bv == m, bi, BIG)
                lsel = jnp.full((L,), jnp.min(cand), _i32)
                cntv = jnp.full((L,), cnt2, _i32)
                plsc.store_scatter(sel, [cntv], lsel, mask=lane0)
                bx1 = plsc.load_gather(cx1, [lsel])
                by1 = plsc.load_gather(cy1, [lsel])
                bx2 = plsc.load_gather(cx2, [lsel])
                by2 = plsc.load_gather(cy2, [lsel])
                bar = plsc.load_gather(car, [lsel])

                def sup(j, carry2):
                    bv2, bi2 = carry2
                    sl = pl.ds(j * L, L)
                    s = cs[sl]
                    inter = (jnp.maximum(
                        jnp.minimum(bx2, cx2[sl]) - jnp.maximum(bx1, cx1[sl]),
                        _f32(0.0))
                        * jnp.maximum(
                        jnp.minimum(by2, cy2[sl]) - jnp.maximum(by1, cy1[sl]),
                        _f32(0.0)))
                    den = (bar + car[sl]) - inter + _f32(1e-9)
                    s = jnp.where(inter / den > _f32(0.5), _f32(-1.0), s)
                    cs[sl] = s
                    li = j * L + iota
                    upd = s > bv2
                    return (jnp.where(upd, s, bv2), jnp.where(upd, li, bi2))

                bv2, bi2 = lax.fori_loop(
                    0, mv, sup,
                    (jnp.full((L,), -2.0, _f32), jnp.zeros((L,), _i32)))
                return cnt2 + 1, jnp.max(bv2), bv2, bi2

            cnt1, _, _, _ = lax.while_loop(
                cond, body, (cnt, jnp.max(bv0), bv0, bi0))

            def ep(j, _):
                kv = j * L + iota
                ok = (kv >= cnt) & (kv < cnt1)
                ls = jnp.where(ok, sel[pl.ds(j * L, L)], 0)
                nsl = plsc.load_gather(cidx, [ls])
                plsc.store_scatter(ks, [kv], plsc.load_gather(sv, [nsl]),
                                   mask=ok)
                plsc.store_scatter(kx1, [kv], plsc.load_gather(gx1, [nsl]),
                                   mask=ok)
                plsc.store_scatter(ky1, [kv], plsc.load_gather(gy1, [nsl]),
                                   mask=ok)
                plsc.store_scatter(kx2, [kv], plsc.load_gather(gx2, [nsl]),
                                   mask=ok)
                plsc.store_scatter(ky2, [kv], plsc.load_gather(gy2, [nsl]),
                                   mask=ok)
                plsc.store_scatter(kn, [kv], nsl, mask=ok)
                plsc.store_scatter(kbx1, [kv], plsc.load_gather(cx1, [ls]),
                                   mask=ok)
                plsc.store_scatter(kby1, [kv], plsc.load_gather(cy1, [ls]),
                                   mask=ok)
                plsc.store_scatter(kbx2, [kv], plsc.load_gather(cx2, [ls]),
                                   mask=ok)
                plsc.store_scatter(kby2, [kv], plsc.load_gather(cy2, [ls]),
                                   mask=ok)
                plsc.store_scatter(kbar, [kv], plsc.load_gather(car, [ls]),
                                   mask=ok)
                return 0

            lax.fori_loop(lax.div(cnt, _i32(L)),
                          lax.div(cnt1 + (L - 1), _i32(L)), ep, 0)
            return cnt1, bptr, rem

        lax.while_loop(outer_cond, outer_body, (_i32(0), _i32(NB - 1), vcnt))

        pltpu.sync_copy(ks, osc_hbm.at[wid])
        pltpu.sync_copy(kx1, ox1_hbm.at[wid])
        pltpu.sync_copy(ky1, oy1_hbm.at[wid])
        pltpu.sync_copy(kx2, ox2_hbm.at[wid])
        pltpu.sync_copy(ky2, oy2_hbm.at[wid])
        pltpu.sync_copy(kn, on_hbm.at[wid])


@functools.partial(
    pl.kernel,
    out_type=[jax.ShapeDtypeStruct((OPAD * 5,), _f32),
              jax.ShapeDtypeStruct((OPAD,), _i32)],
    mesh=_mesh,
    compiler_params=pltpu.CompilerParams(needs_layout_passes=False, use_tc_tiling_on_sc=False),
    scratch_types=[
        pltpu.VMEM((C * KCAP + L,), _f32),
        pltpu.VMEM((C * KCAP + L,), _f32),
        pltpu.VMEM((C * KCAP + L,), _f32),
        pltpu.VMEM((C * KCAP + L,), _f32),
        pltpu.VMEM((C * KCAP + L,), _f32),
        pltpu.VMEM((C * KCAP + L,), _i32),
        pltpu.VMEM((2 * L,), _f32),
        pltpu.VMEM((2 * L,), _i32),
        pltpu.SMEM((2 * L,), _i32),
        pltpu.VMEM((OPAD * 5,), _f32),
        pltpu.VMEM((OPAD,), _i32),
        pltpu.VMEM((OPAD,), _i32),
    ],
)
def _merge_kernel(sc_hbm, x1_hbm, y1_hbm, x2_hbm, y2_hbm, n_hbm,
                  dets_hbm, labels_hbm,
                  vsc, vx1, vy1, vx2, vy2, vn, hs, hn, hp, dv, lv, sb):
    wid = _wid()
    iota = lax.iota(_i32, L)
    lane0 = iota == 0

    @pl.when(wid == 0)
    def _():
        pltpu.sync_copy(sc_hbm, vsc.at[pl.ds(0, C * KCAP)])
        pltpu.sync_copy(x1_hbm, vx1.at[pl.ds(0, C * KCAP)])
        pltpu.sync_copy(y1_hbm, vy1.at[pl.ds(0, C * KCAP)])
        pltpu.sync_copy(x2_hbm, vx2.at[pl.ds(0, C * KCAP)])
        pltpu.sync_copy(y2_hbm, vy2.at[pl.ds(0, C * KCAP)])
        pltpu.sync_copy(n_hbm, vn.at[pl.ds(0, C * KCAP)])

        for half in range(2):
            cv = iota + half * L
            cidx = jnp.minimum(cv, C - 1) * KCAP
            h = plsc.load_gather(vsc, [cidx])
            hs[pl.ds(half * L, L)] = jnp.where(cv < C, h, _f32(-1.0))
            nh = plsc.load_gather(vn, [cidx])
            hn[pl.ds(half * L, L)] = jnp.where(cv < C, nh, _i32(0))

        def pinit(c, _):
            hp[c] = _i32(0)
            return 0

        lax.fori_loop(0, 2 * L, pinit, 0)

        def sinit(k, _):
            sb[pl.ds(k * L, L)] = jnp.full((L,), -1, _i32)
            return 0

        lax.fori_loop(0, OPAD // L, sinit, 0)

        def mbody(k, _):
            h1 = hs[pl.ds(0, L)]
            h2 = hs[pl.ds(L, L)]
            m = jnp.maximum(jnp.max(h1), jnp.max(h2))

            @pl.when(m > _f32(0.0))
            def _():
                n1 = hn[pl.ds(0, L)]
                n2 = hn[pl.ds(L, L)]
                fi1 = jnp.where(h1 == m, n1 * C + iota, BIG)
                fi2 = jnp.where(h2 == m, n2 * C + (iota + L), BIG)
                fi = jnp.minimum(jnp.min(fi1), jnp.min(fi2))
                csel = lax.rem(fi, _i32(C))
                p = hp[csel]
                base = csel * KCAP + p
                _sstore(sb, k, base, lane0)
                pn = p + 1
                hp[csel] = pn
                pc = jnp.minimum(pn, KCAP - 1)
                nxt = _sload(vsc, csel * KCAP + pc)
                _sstore(hs, csel, jnp.where(pn > KCAP - 1, _f32(-1.0), nxt),
                        lane0)
                _sstore(hn, csel, _sload(vn, csel * KCAP + pc), lane0)

            return 0

        lax.fori_loop(0, MAX_NUM, mbody, 0)

        allm = iota >= 0

        def epi(j, _):
            kv = j * L + iota
            b = sb[pl.ds(j * L, L)]
            ok = b >= 0
            bc = jnp.where(ok, b, 0)
            zero = jnp.zeros((L,), _f32)
            plsc.store_scatter(
                dv, [kv * 5],
                jnp.where(ok, plsc.load_gather(vx1, [bc]), zero), mask=allm)
            plsc.store_scatter(
                dv, [kv * 5 + 1],
                jnp.where(ok, plsc.load_gather(vy1, [bc]), zero), mask=allm)
            plsc.store_scatter(
                dv, [kv * 5 + 2],
                jnp.where(ok, plsc.load_gather(vx2, [bc]), zero), mask=allm)
            plsc.store_scatter(
                dv, [kv * 5 + 3],
                jnp.where(ok, plsc.load_gather(vy2, [bc]), zero), mask=allm)
            plsc.store_scatter(
                dv, [kv * 5 + 4],
                jnp.where(ok, plsc.load_gather(vsc, [bc]), zero), mask=allm)
            lv[pl.ds(j * L, L)] = jnp.where(
                ok, lax.shift_right_logical(bc, 7), -1)
            return 0

        lax.fori_loop(0, OPAD // L, epi, 0)

        pltpu.sync_copy(dv, dets_hbm)
        pltpu.sync_copy(lv, labels_hbm)


def kernel(multi_bboxes, multi_scores):
    pad = ((0, 0), (0, NP - N))
    bbt = jnp.pad(multi_bboxes.T, pad)
    st = jnp.pad(multi_scores.T[:C], pad)
    k_sc, k_x1, k_y1, k_x2, k_y2, k_n = _nms_kernel(
        st.reshape(-1), bbt.reshape(-1))
    dets_pad, labels_pad = _merge_kernel(
        k_sc.reshape(-1), k_x1.reshape(-1), k_y1.reshape(-1),
        k_x2.reshape(-1), k_y2.reshape(-1), k_n.reshape(-1))
    dets = dets_pad.reshape(OPAD, 5)[:MAX_NUM]
    labels = labels_pad[:MAX_NUM]
    return dets, labels


# speedup vs baseline: 1.1571x; 1.0029x over previous
import functools

import jax
import jax.numpy as jnp
import numpy as np
from jax import lax
from jax.experimental import pallas as pl
from jax.experimental.pallas import tpu as pltpu
from jax.experimental.pallas import tpu_sc as plsc

SCORE_THR = 0.05
MAX_NUM = 100
N = 5000
C = 20
L = 16
NP = 5008
NV = NP // L
KCAP = 128
OPAD = 128
NCORES = 2
NSUB = 16
BIG = np.int32(1 << 30)

_mesh = plsc.VectorSubcoreMesh(
    core_axis_name="c", subcore_axis_name="s",
    num_cores=NCORES, num_subcores=NSUB)

_f32 = np.float32
_i32 = np.int32


def _wid():
    return lax.axis_index("s") * NCORES + lax.axis_index("c")


def _sload(ref, idx):
    return ref[pl.ds(idx, L)][0]


def _sstore(ref, idx, val, lane0):
    plsc.store_scatter(
        ref, [jnp.full((L,), idx, _i32)], jnp.full((L,), val), mask=lane0)


NB = 256
CH = 112
NTASK = 2 * C


@functools.partial(
    pl.kernel,
    out_type=[jax.ShapeDtypeStruct((C, KCAP), _f32)] * 5
    + [jax.ShapeDtypeStruct((C, KCAP), _i32)],
    mesh=_mesh,
    compiler_params=pltpu.CompilerParams(needs_layout_passes=False, use_tc_tiling_on_sc=False),
    scratch_types=[
        pltpu.VMEM((NP,), _f32),
        pltpu.VMEM((NP + L,), _f32),
        pltpu.VMEM((NP + L,), _f32),
        pltpu.VMEM((NP + L,), _f32),
        pltpu.VMEM((NP + L,), _f32),
        pltpu.VMEM((NP,), _i32),
        pltpu.VMEM((NB * L,), _i32),
        pltpu.VMEM((NSUB * L,), _f32),
        pltpu.VMEM((NP,), _f32),
        pltpu.VMEM((L,), _f32),
        pltpu.VMEM_SHARED((NSUB * L,), _f32),
        pltpu.VMEM((NP + L,), _f32),
        pltpu.VMEM((NP + L,), _i32),
        pltpu.VMEM((NP + L,), _f32),
        pltpu.VMEM((NP + L,), _f32),
        pltpu.VMEM((NP + L,), _f32),
        pltpu.VMEM((NP + L,), _f32),
        pltpu.VMEM((NP + L,), _f32),
        pltpu.VMEM((KCAP,), _f32),
        pltpu.VMEM((KCAP,), _f32),
        pltpu.VMEM((KCAP,), _f32),
        pltpu.VMEM((KCAP,), _f32),
        pltpu.VMEM((KCAP,), _f32),
        pltpu.VMEM((KCAP,), _i32),
        pltpu.VMEM((KCAP,), _f32),
        pltpu.VMEM((KCAP,), _f32),
        pltpu.VMEM((KCAP,), _f32),
        pltpu.VMEM((KCAP,), _f32),
        pltpu.VMEM((KCAP,), _f32),
        pltpu.VMEM((KCAP,), _i32),
    ],
)
def _nms_kernel(s_hbm, bb_hbm,
                osc_hbm, ox1_hbm, oy1_hbm, ox2_hbm, oy2_hbm, on_hbm,
                sv, gx1, gy1, gx2, gy2,
                bkt, hist, mbuf, slab, accb, shm, cs, cidx,
                cx1, cy1, cx2, cy2, car,
                ks, kx1, ky1, kx2, ky2, kn, kbx1, kby1, kbx2, kby2, kbar,
                sel):
    wid = _wid()
    iota = lax.iota(_i32, L)
    lane0 = iota == 0
    ones = jnp.ones((L,), _i32)

    sid = lax.axis_index("s")

    def mslab(j, acc):
        return jnp.maximum(acc, slab[pl.ds(j * L, L)])

    def mtask(t, acc):
        tt = jnp.minimum(sid + t * NSUB, NTASK - 1)
        r = 4 * lax.div(tt, 2) + 2 + lax.rem(tt, 2)
        pltpu.sync_copy(bb_hbm.at[pl.ds(r * NP, NP)], slab)
        return lax.fori_loop(0, NV, mslab, acc)

    acc = lax.fori_loop(0, (NTASK + NSUB - 1) // NSUB, mtask,
                        jnp.full((L,), -1e30, _f32))
    accb[...] = acc
    pltpu.sync_copy(accb, shm.at[pl.ds(sid * L, L)])
    plsc.subcore_barrier()
    pltpu.sync_copy(shm, mbuf)

    def mbody(j, acc):
        return jnp.maximum(acc, mbuf[pl.ds(j * L, L)])

    mxv = lax.fori_loop(0, NSUB, mbody, jnp.full((L,), -1e30, _f32))
    gmax = jnp.max(mxv)

    @pl.when(wid < C)
    def _():
        pltpu.sync_copy(s_hbm.at[pl.ds(wid * NP, NP)], sv)
        base = 4 * wid * NP
        pltpu.sync_copy(bb_hbm.at[pl.ds(base, NP)], gx1.at[pl.ds(0, NP)])
        pltpu.sync_copy(bb_hbm.at[pl.ds(base + NP, NP)],
                        gy1.at[pl.ds(0, NP)])
        pltpu.sync_copy(bb_hbm.at[pl.ds(base + 2 * NP, NP)],
                        gx2.at[pl.ds(0, NP)])
        pltpu.sync_copy(bb_hbm.at[pl.ds(base + 3 * NP, NP)],
                        gy2.at[pl.ds(0, NP)])
        off = wid.astype(_f32) * (gmax + _f32(1.0))

        def hz(j, _):
            hist[pl.ds(j * L, L)] = jnp.zeros((L,), _i32)
            return 0

        lax.fori_loop(0, NB, hz, 0)

        def p1(j, vcnt):
            sl = pl.ds(j * L, L)
            s = sv[sl]
            s = jnp.where(s > _f32(SCORE_THR), s, _f32(-1.0))
            sv[sl] = s
            valid = s > _f32(0.0)
            b = jnp.clip((s * _f32(NB)).astype(_i32), 0, NB - 1)
            bkt[sl] = jnp.where(valid, b, -1)
            plsc.addupdate_scatter(hist, [b * L + iota], ones, mask=valid)
            return vcnt + plsc.all_reduce_population_count(valid)[0]

        vcnt = lax.fori_loop(0, NV, p1, _i32(0))

        def ki(j, _):
            sl = pl.ds(j * L, L)
            ks[sl] = jnp.full((L,), -1.0, _f32)
            kx1[sl] = jnp.zeros((L,), _f32)
            ky1[sl] = jnp.zeros((L,), _f32)
            kx2[sl] = jnp.zeros((L,), _f32)
            ky2[sl] = jnp.zeros((L,), _f32)
            kn[sl] = jnp.zeros((L,), _i32)
            return 0

        lax.fori_loop(0, KCAP // L, ki, 0)

        def outer_cond(st):
            cnt, bp, rem = st
            return (cnt < MAX_NUM) & (rem > 0) & (bp >= 0)

        def outer_body(st):
            cnt, bp, rem = st

            def wcond(ws):
                acc, bptr = ws
                return (acc < CH) & (bptr >= 0)

            def wbody(ws):
                acc, bptr = ws
                cb = jnp.sum(hist[pl.ds(bptr * L, L)])
                return acc + cb, bptr - 1

            acc, bptr = lax.while_loop(wcond, wbody, (_i32(0), bp))
            b_lo = bptr + 1
            rem = rem - acc

            def coll(j, wp):
                sl = pl.ds(j * L, L)
                b = bkt[sl]
                msk = (b >= b_lo) & (b <= bp)
                plsc.store_compressed(cs.at[pl.ds(wp, L)], sv[sl], mask=msk)
                plsc.store_compressed(cidx.at[pl.ds(wp, L)],
                                      j * L + iota, mask=msk)
                return wp + plsc.all_reduce_population_count(msk)[0]

            m_sz = lax.fori_loop(0, NV, coll, _i32(0))
            cs[pl.ds(m_sz, L)] = jnp.full((L,), -1.0, _f32)
            cidx[pl.ds(m_sz, L)] = jnp.zeros((L,), _i32)
            mv = lax.div(m_sz + (L - 1), _i32(L))

            def cg(j, _):
                sl = pl.ds(j * L, L)
                iv = cidx[sl]
                x1 = plsc.load_gather(gx1, [iv]) + off
                y1 = plsc.load_gather(gy1, [iv]) + off
                x2 = plsc.load_gather(gx2, [iv]) + off
                y2 = plsc.load_gather(gy2, [iv]) + off
                cx1[sl] = x1
                cy1[sl] = y1
                cx2[sl] = x2
                cy2[sl] = y2
                car[sl] = (x2 - x1) * (y2 - y1)
                return 0

            lax.fori_loop(0, mv, cg, 0)

            def kchk(k, _):
                bx1 = _sload(kbx1, k)
                by1 = _sload(kby1, k)
                bx2 = _sload(kbx2, k)
                by2 = _sload(kby2, k)
                bar = _sload(kbar, k)

                def kchk_j(j, _2):
                    sl = pl.ds(j * L, L)
                    inter = (jnp.maximum(
                        jnp.minimum(bx2, cx2[sl]) - jnp.maximum(bx1, cx1[sl]),
                        _f32(0.0))
                        * jnp.maximum(
                        jnp.minimum(by2, cy2[sl]) - jnp.maximum(by1, cy1[sl]),
                        _f32(0.0)))
                    den = (bar + car[sl]) - inter + _f32(1e-9)
                    cs[sl] = jnp.where(inter / den > _f32(0.5), _f32(-1.0),
                                       cs[sl])
                    return 0

                lax.fori_loop(0, mv, kchk_j, 0)
                return 0

            lax.fori_loop(0, cnt, kchk, 0)

            def am(j, carry):
                bv, bi = carry
                s = cs[pl.ds(j * L, L)]
                li = j * L + iota
                upd = s > bv
                return jnp.where(upd, s, bv), jnp.where(upd, li, bi)

            bv0, bi0 = lax.fori_loop(
                0, mv, am,
                (jnp.full((L,), -2.0, _f32), jnp.zeros((L,), _i32)))

            def cond(carry):
                cnt2, m, _, _ = carry
                return (cnt2 < MAX_NUM) & (m > _f32(0.0))

            def body(carry):
                cnt2, m, bv, bi = carry
                cand = jnp.where(bv == m, bi, BIG)
                lsel = jnp.full((L,), jnp.min(cand), _i32)
                cntv = jnp.full((L,), cnt2, _i32)
                plsc.store_scatter(sel, [cntv], lsel, mask=lane0)
                bx1 = plsc.load_gather(cx1, [lsel])
                by1 = plsc.load_gather(cy1, [lsel])
                bx2 = plsc.load_gather(cx2, [lsel])
                by2 = plsc.load_gather(cy2, [lsel])
                bar = plsc.load_gather(car, [lsel])

                def sup(j, carry2):
                    bv2, bi2 = carry2
                    sl = pl.ds(j * L, L)
                    s = cs[sl]
                    inter = (jnp.maximum(
                        jnp.minimum(bx2, cx2[sl]) - jnp.maximum(bx1, cx1[sl]),
                        _f32(0.0))
                        * jnp.maximum(
                        jnp.minimum(by2, cy2[sl]) - jnp.maximum(by1, cy1[sl]),
                        _f32(0.0)))
                    den = (bar + car[sl]) - inter + _f32(1e-9)
                    s = jnp.where(inter / den > _f32(0.5), _f32(-1.0), s)
                    cs[sl] = s
                    li = j * L + iota
                    upd = s > bv2
                    return (jnp.where(upd, s, bv2), jnp.where(upd, li, bi2))

                bv2, bi2 = lax.fori_loop(
                    0, mv, sup,
                    (jnp.full((L,), -2.0, _f32), jnp.zeros((L,), _i32)))
                return cnt2 + 1, jnp.max(bv2), bv2, bi2

            cnt1, _, _, _ = lax.while_loop(
                cond, body, (cnt, jnp.max(bv0), bv0, bi0))

            def ep(j, _):
                kv = j * L + iota
                ok = (kv >= cnt) & (kv < cnt1)
                ls = jnp.where(ok, sel[pl.ds(j * L, L)], 0)
                nsl = plsc.load_gather(cidx, [ls])
                plsc.store_scatter(ks, [kv], plsc.load_gather(sv, [nsl]),
                                   mask=ok)
                plsc.store_scatter(kx1, [kv], plsc.load_gather(gx1, [nsl]),
                                   mask=ok)
                plsc.store_scatter(ky1, [kv], plsc.load_gather(gy1, [nsl]),
                                   mask=ok)
                plsc.store_scatter(kx2, [kv], plsc.load_gather(gx2, [nsl]),
                                   mask=ok)
                plsc.store_scatter(ky2, [kv], plsc.load_gather(gy2, [nsl]),
                                   mask=ok)
                plsc.store_scatter(kn, [kv], nsl, mask=ok)
                plsc.store_scatter(kbx1, [kv], plsc.load_gather(cx1, [ls]),
                                   mask=ok)
                plsc.store_scatter(kby1, [kv], plsc.load_gather(cy1, [ls]),
                                   mask=ok)
                plsc.store_scatter(kbx2, [kv], plsc.load_gather(cx2, [ls]),
                                   mask=ok)
                plsc.store_scatter(kby2, [kv], plsc.load_gather(cy2, [ls]),
                                   mask=ok)
                plsc.store_scatter(kbar, [kv], plsc.load_gather(car, [ls]),
                                   mask=ok)
                return 0

            lax.fori_loop(lax.div(cnt, _i32(L)),
                          lax.div(cnt1 + (L - 1), _i32(L)), ep, 0)
            return cnt1, bptr, rem

        lax.while_loop(outer_cond, outer_body, (_i32(0), _i32(NB - 1), vcnt))

        pltpu.sync_copy(ks, osc_hbm.at[wid])
        pltpu.sync_copy(kx1, ox1_hbm.at[wid])
        pltpu.sync_copy(ky1, oy1_hbm.at[wid])
        pltpu.sync_copy(kx2, ox2_hbm.at[wid])
        pltpu.sync_copy(ky2, oy2_hbm.at[wid])
        pltpu.sync_copy(kn, on_hbm.at[wid])


@functools.partial(
    pl.kernel,
    out_type=[jax.ShapeDtypeStruct((OPAD * 5,), _f32),
              jax.ShapeDtypeStruct((OPAD,), _i32)],
    mesh=_mesh,
    compiler_params=pltpu.CompilerParams(needs_layout_passes=False, use_tc_tiling_on_sc=False),
    scratch_types=[
        pltpu.VMEM((C * KCAP + L,), _f32),
        pltpu.VMEM((C * KCAP + L,), _f32),
        pltpu.VMEM((C * KCAP + L,), _f32),
        pltpu.VMEM((C * KCAP + L,), _f32),
        pltpu.VMEM((C * KCAP + L,), _f32),
        pltpu.VMEM((C * KCAP + L,), _i32),
        pltpu.VMEM((2 * L,), _f32),
        pltpu.VMEM((2 * L,), _i32),
        pltpu.SMEM((2 * L,), _i32),
        pltpu.VMEM((OPAD * 5,), _f32),
        pltpu.VMEM((OPAD,), _i32),
        pltpu.VMEM((OPAD,), _i32),
    ],
)
def _merge_kernel(sc_hbm, x1_hbm, y1_hbm, x2_hbm, y2_hbm, n_hbm,
                  dets_hbm, labels_hbm,
                  vsc, vx1, vy1, vx2, vy2, vn, hs, hn, hp, dv, lv, sb):
    wid = _wid()
    iota = lax.iota(_i32, L)
    lane0 = iota == 0

    @pl.when(wid == 0)
    def _():
        pltpu.sync_copy(sc_hbm, vsc.at[pl.ds(0, C * KCAP)])
        pltpu.sync_copy(x1_hbm, vx1.at[pl.ds(0, C * KCAP)])
        pltpu.sync_copy(y1_hbm, vy1.at[pl.ds(0, C * KCAP)])
        pltpu.sync_copy(x2_hbm, vx2.at[pl.ds(0, C * KCAP)])
        pltpu.sync_copy(y2_hbm, vy2.at[pl.ds(0, C * KCAP)])
        pltpu.sync_copy(n_hbm, vn.at[pl.ds(0, C * KCAP)])

        for half in range(2):
            cv = iota + half * L
            cidx = jnp.minimum(cv, C - 1) * KCAP
            h = plsc.load_gather(vsc, [cidx])
            hs[pl.ds(half * L, L)] = jnp.where(cv < C, h, _f32(-1.0))
            nh = plsc.load_gather(vn, [cidx])
            hn[pl.ds(half * L, L)] = jnp.where(cv < C, nh, _i32(0))

        def pinit(c, _):
            hp[c] = _i32(0)
            return 0

        lax.fori_loop(0, 2 * L, pinit, 0)

        def sinit(k, _):
            sb[pl.ds(k * L, L)] = jnp.full((L,), -1, _i32)
            return 0

        lax.fori_loop(0, OPAD // L, sinit, 0)

        def mbody(k, _):
            h1 = hs[pl.ds(0, L)]
            h2 = hs[pl.ds(L, L)]
            m = jnp.maximum(jnp.max(h1), jnp.max(h2))

            @pl.when(m > _f32(0.0))
            def _():
                n1 = hn[pl.ds(0, L)]
                n2 = hn[pl.ds(L, L)]
                fi1 = jnp.where(h1 == m, n1 * C + iota, BIG)
                fi2 = jnp.where(h2 == m, n2 * C + (iota + L), BIG)
                fi = jnp.minimum(jnp.min(fi1), jnp.min(fi2))
                csel = lax.rem(fi, _i32(C))
                p = hp[csel]
                base = csel * KCAP + p
                _sstore(sb, k, base, lane0)
                pn = p + 1
                hp[csel] = pn
                pc = jnp.minimum(pn, KCAP - 1)
                nxt = _sload(vsc, csel * KCAP + pc)
                _sstore(hs, csel, jnp.where(pn > KCAP - 1, _f32(-1.0), nxt),
                        lane0)
                _sstore(hn, csel, _sload(vn, csel * KCAP + pc), lane0)

            return 0

        lax.fori_loop(0, MAX_NUM, mbody, 0)

        allm = iota >= 0

        def epi(j, _):
            kv = j * L + iota
            b = sb[pl.ds(j * L, L)]
            ok = b >= 0
            bc = jnp.where(ok, b, 0)
            zero = jnp.zeros((L,), _f32)
            plsc.store_scatter(
                dv, [kv * 5],
                jnp.where(ok, plsc.load_gather(vx1, [bc]), zero), mask=allm)
            plsc.store_scatter(
                dv, [kv * 5 + 1],
                jnp.where(ok, plsc.load_gather(vy1, [bc]), zero), mask=allm)
            plsc.store_scatter(
                dv, [kv * 5 + 2],
                jnp.where(ok, plsc.load_gather(vx2, [bc]), zero), mask=allm)
            plsc.store_scatter(
                dv, [kv * 5 + 3],
                jnp.where(ok, plsc.load_gather(vy2, [bc]), zero), mask=allm)
            plsc.store_scatter(
                dv, [kv * 5 + 4],
                jnp.where(ok, plsc.load_gather(vsc, [bc]), zero), mask=allm)
            lv[pl.ds(j * L, L)] = jnp.where(
                ok, lax.shift_right_logical(bc, 7), -1)
            return 0

        lax.fori_loop(0, OPAD // L, epi, 0)

        pltpu.sync_copy(dv, dets_hbm)
        pltpu.sync_copy(lv, labels_hbm)


def kernel(multi_bboxes, multi_scores):
    pad = ((0, 0), (0, NP - N))
    bbt = jnp.pad(multi_bboxes.T, pad)
    st = jnp.pad(multi_scores.T[:C], pad)
    k_sc, k_x1, k_y1, k_x2, k_y2, k_n = _nms_kernel(
        st.reshape(-1), bbt.reshape(-1))
    dets_pad, labels_pad = _merge_kernel(
        k_sc.reshape(-1), k_x1.reshape(-1), k_y1.reshape(-1),
        k_x2.reshape(-1), k_y2.reshape(-1), k_n.reshape(-1))
    dets = dets_pad.reshape(OPAD, 5)[:MAX_NUM]
    labels = labels_pad[:MAX_NUM]
    return dets, labels
